# baseline (speedup 1.0000x reference)
import functools

import jax
import jax.numpy as jnp
from jax import lax
from jax.experimental import pallas as pl
from jax.experimental.pallas import tpu as pltpu
from jax.experimental.pallas import tpu_sc as plsc

_B, _N, _C, _A = 16, 8192, 80, 64
_NV = 48


def _sc_scatter_rows(ann_t):
    mesh = plsc.VectorSubcoreMesh(core_axis_name="c", subcore_axis_name="s")

    @functools.partial(
        pl.kernel,
        mesh=mesh,
        out_type=jax.ShapeDtypeStruct((_B, _NV), jnp.int32),
        scratch_types=[
            pltpu.VMEM((_A,), jnp.float32),
            pltpu.VMEM((_A,), jnp.float32),
            pltpu.VMEM((_NV,), jnp.int32),
        ],
        compiler_params=pltpu.CompilerParams(use_tc_tiling_on_sc=False,
                                             needs_layout_passes=False),
    )
    def k(ann_hbm, pi_hbm, s_v, e_v, idx_v):
        cid = lax.axis_index("c")
        sid = lax.axis_index("s")
        wid = sid * 2 + cid

        @pl.when(wid < _B)
        def _():
            b = wid
            pltpu.sync_copy(ann_hbm.at[b, 0], s_v)
            pltpu.sync_copy(ann_hbm.at[b, 1], e_v)
            for j in range(_NV // 16):
                sv = s_v[pl.ds(j * 16, 16)]
                ev = e_v[pl.ds(j * 16, 16)]
                tp = ((sv + ev) * 0.5) * 100.0
                idx_v[pl.ds(j * 16, 16)] = tp.astype(jnp.int32)
            pltpu.sync_copy(idx_v, pi_hbm.at[b])

    return k(ann_t)


def _smooth_l1(d):
    return jnp.where(d <= 1.0, 0.5 * d * d, d - 0.5)


def _tc_body(pi_ref, ct_ref, rt_ref, ann_ref, out_cls_ref, out_reg_ref,
             acc_ref, rcols_ref, sem):
    b = pl.program_id(0)

    @pl.when(b == 0)
    def _init():
        acc_ref[0] = 0.0
        acc_ref[1] = 0.0

    copies = []
    for kk in range(_NV):
        idx = pi_ref[b, kk]
        al = pl.multiple_of((idx // 128) * 128, 128)
        cp = pltpu.make_async_copy(
            rt_ref.at[b, :, pl.ds(al, 128)],
            rcols_ref.at[kk],
            sem,
        )
        cp.start()
        copies.append(cp)

    xc = jnp.clip(ct_ref[0], 0.0001, 1.0 - 0.0001)
    part = jnp.sum(xc * xc * jnp.log(1.0 - xc))

    ann = ann_ref[0]
    s = ann[0:1, :]
    e = ann[1:2, :]
    cl = ann[2:3, :]
    pii = (((s + e) * 0.5) * 100.0).astype(jnp.int32)
    pif = pii.astype(jnp.float32)
    clf = cl.astype(jnp.int32).astype(jnp.float32)
    lane = lax.broadcasted_iota(jnp.int32, (1, _A), 1)
    validl = (lane < _NV).astype(jnp.float32)
    keyl = pif * float(_C) + clf

    ident = (lax.broadcasted_iota(jnp.int32, (_A, _A), 0)
             == lax.broadcasted_iota(jnp.int32, (_A, _A), 1)).astype(jnp.float32)

    def to_sub(v):
        return jnp.sum(ident * v, axis=1, keepdims=True)

    key_s = to_sub(keyl)
    pi_s = to_sub(pif)
    valid_s = to_sub(validl)
    js = lax.broadcasted_iota(jnp.int32, (_A, _A), 0)
    ks = lax.broadcasted_iota(jnp.int32, (_A, _A), 1)
    dup = (key_s == keyl) & (js < ks) & (valid_s > 0.0)
    dup_l = jnp.sum(dup.astype(jnp.float32), axis=0, keepdims=True)
    m_full = validl * (dup_l == 0.0).astype(jnp.float32)
    cntm = (pi_s == pif) & (valid_s > 0.0)
    cnt_full = jnp.sum(cntm.astype(jnp.float32), axis=0, keepdims=True)

    cl_s = to_sub(clf)[:_NV, :].astype(jnp.int32)
    pi_si = pi_s[:_NV, :].astype(jnp.int32)

    ohc = (lax.broadcasted_iota(jnp.int32, (_NV, _C), 1)
           == cl_s).astype(jnp.float32)
    oh48 = lax.broadcasted_iota(jnp.int32, (_NV, _N), 1) == pi_si
    rowsc = jnp.dot(ohc, xc, preferred_element_type=jnp.float32,
                    precision=lax.Precision.HIGHEST)
    cg = jnp.sum(jnp.where(oh48, rowsc, 0.0), axis=1, keepdims=True)

    l1 = -0.25 * (1.0 - cg) * (1.0 - cg) * jnp.log(cg)
    l0 = -0.75 * cg * cg * jnp.log(1.0 - cg)
    m_s = to_sub(m_full)[:_NV, :]
    cls_corr = jnp.sum(m_s * (l1 - l0))

    for cp in copies:
        cp.wait()
    rw = rcols_ref[...]
    off3 = jnp.reshape(pi_si % 128, (_NV, 1, 1))
    lmask = lax.broadcasted_iota(jnp.int32, (_NV, _C, 128), 2) == off3
    rcolsT = jnp.sum(jnp.where(lmask, rw, 0.0), axis=2)
    reg_base = jnp.sum(_smooth_l1(jnp.abs(rcolsT)))
    gmask = lax.broadcasted_iota(jnp.int32, (_NV, _C), 1) == cl_s
    g = jnp.sum(jnp.where(gmask, rcolsT, 0.0), axis=1, keepdims=True)
    cnt_s = to_sub(cnt_full)[:_NV, :]
    reg_corr = jnp.sum(m_s * cnt_s * (_smooth_l1(jnp.abs(1.0 - g))
                                      - _smooth_l1(jnp.abs(g))))

    acc_ref[0] = acc_ref[0] + (-0.75) * part + cls_corr
    acc_ref[1] = acc_ref[1] + reg_base + reg_corr

    @pl.when(b == _B - 1)
    def _fin():
        out_cls_ref[0, 0] = acc_ref[0] / float(_NV * _B)
        out_reg_ref[0, 0] = acc_ref[1] / float(_NV * _C * _B)


def _tc_call_kwargs():
    return dict(
        grid=(_B,),
        in_specs=[
            pl.BlockSpec(memory_space=pltpu.SMEM),
            pl.BlockSpec((1, _C, _N), lambda b: (b, 0, 0)),
            pl.BlockSpec(memory_space=pl.ANY),
            pl.BlockSpec((1, 3, _A), lambda b: (b, 0, 0)),
        ],
        out_specs=[
            pl.BlockSpec(memory_space=pltpu.SMEM),
            pl.BlockSpec(memory_space=pltpu.SMEM),
        ],
        out_shape=[
            jax.ShapeDtypeStruct((1, 1), jnp.float32),
            jax.ShapeDtypeStruct((1, 1), jnp.float32),
        ],
        scratch_shapes=[
            pltpu.SMEM((2,), jnp.float32),
            pltpu.VMEM((_NV, _C, 128), jnp.float32),
            pltpu.SemaphoreType.DMA,
        ],
    )


def kernel(classifications, regressions, annotations):
    ann_t = annotations.transpose(0, 2, 1)
    pi = _sc_scatter_rows(ann_t)

    ct = classifications.transpose(0, 2, 1)
    rt = regressions.transpose(0, 2, 1)
    out_c, out_r = pl.pallas_call(_tc_body, **_tc_call_kwargs())(
        pi, ct, rt, ann_t)
    return out_c.reshape(1), out_r.reshape(1)


# speedup vs baseline: 1.1512x; 1.1512x over previous
import functools

import jax
import jax.numpy as jnp
from jax import lax
from jax.experimental import pallas as pl
from jax.experimental.pallas import tpu as pltpu
from jax.experimental.pallas import tpu_sc as plsc

_B, _N, _C, _A = 16, 8192, 80, 64
_NV = 48


def _sc_scatter_counts(ann_t, zeros_n):
    mesh = plsc.VectorSubcoreMesh(core_axis_name="c", subcore_axis_name="s")

    @functools.partial(
        pl.kernel,
        mesh=mesh,
        out_type=jax.ShapeDtypeStruct((_B, 1, _N), jnp.float32),
        scratch_types=[
            pltpu.VMEM((_A,), jnp.float32),
            pltpu.VMEM((_A,), jnp.float32),
            pltpu.VMEM((_N,), jnp.float32),
        ],
        compiler_params=pltpu.CompilerParams(use_tc_tiling_on_sc=False,
                                             needs_layout_passes=False),
    )
    def k(ann_hbm, zeros_hbm, w_hbm, s_v, e_v, w_v):
        cid = lax.axis_index("c")
        sid = lax.axis_index("s")
        wid = sid * 2 + cid

        @pl.when(wid < _B)
        def _():
            b = wid
            pltpu.sync_copy(zeros_hbm, w_v)
            pltpu.sync_copy(ann_hbm.at[b, 0], s_v)
            pltpu.sync_copy(ann_hbm.at[b, 1], e_v)
            ones = jnp.ones((16,), jnp.float32)
            for j in range(_NV // 16):
                sv = s_v[pl.ds(j * 16, 16)]
                ev = e_v[pl.ds(j * 16, 16)]
                tp = ((sv + ev) * 0.5) * 100.0
                plsc.addupdate_scatter(w_v, [tp.astype(jnp.int32)], ones)
            pltpu.sync_copy(w_v, w_hbm.at[b, 0])

    return k(ann_t, zeros_n)


def _smooth_l1(d):
    return jnp.where(d <= 1.0, 0.5 * d * d, d - 0.5)


def _tc_body(ct_ref, rt_ref, ann_ref, sums_ref, scol_ref, acc_ref):
    b = pl.program_id(0)

    @pl.when(b == 0)
    def _init():
        acc_ref[0] = 0.0
        acc_ref[1] = 0.0

    xc = jnp.clip(ct_ref[0], 0.0001, 1.0 - 0.0001)
    part = jnp.sum(xc * xc * jnp.log(1.0 - xc))

    ann = ann_ref[0]
    s = ann[0:1, :]
    e = ann[1:2, :]
    cl = ann[2:3, :]
    pii = (((s + e) * 0.5) * 100.0).astype(jnp.int32)
    pif = pii.astype(jnp.float32)
    clf = cl.astype(jnp.int32).astype(jnp.float32)
    lane = lax.broadcasted_iota(jnp.int32, (1, _A), 1)
    validl = (lane < _NV).astype(jnp.float32)
    keyl = pif * float(_C) + clf

    ident = (lax.broadcasted_iota(jnp.int32, (_A, _A), 0)
             == lax.broadcasted_iota(jnp.int32, (_A, _A), 1)).astype(jnp.float32)

    def to_sub(v):
        return jnp.sum(ident * v, axis=1, keepdims=True)

    key_s = to_sub(keyl)
    pi_s = to_sub(pif)
    valid_s = to_sub(validl)
    js = lax.broadcasted_iota(jnp.int32, (_A, _A), 0)
    ks = lax.broadcasted_iota(jnp.int32, (_A, _A), 1)
    dup = (key_s == keyl) & (js < ks) & (valid_s > 0.0)
    dup_l = jnp.sum(dup.astype(jnp.float32), axis=0, keepdims=True)
    m_l = validl * (dup_l == 0.0).astype(jnp.float32)
    cntm = (pi_s == pif) & (valid_s > 0.0)
    cnt_l = jnp.sum(cntm.astype(jnp.float32), axis=0, keepdims=True)

    m_s = to_sub(m_l)[:_NV, :]
    cnt_s = to_sub(cnt_l)[:_NV, :]
    cl_s = to_sub(clf)[:_NV, :].astype(jnp.int32)
    pi_si = pi_s[:_NV, :].astype(jnp.int32)

    ohc = (lax.broadcasted_iota(jnp.int32, (_NV, _C), 1)
           == cl_s).astype(jnp.float32)
    oh48 = lax.broadcasted_iota(jnp.int32, (_NV, _N), 1) == pi_si

    rowsc = jnp.dot(ohc, xc, preferred_element_type=jnp.float32,
                    precision=lax.Precision.HIGHEST)
    cg = jnp.sum(jnp.where(oh48, rowsc, 0.0), axis=1, keepdims=True)

    rr = rt_ref[0]
    rowsr = jnp.dot(ohc, rr, preferred_element_type=jnp.float32,
                    precision=lax.Precision.HIGHEST)
    g = jnp.sum(jnp.where(oh48, rowsr, 0.0), axis=1, keepdims=True)

    scol_ref[0] = jnp.sum(_smooth_l1(jnp.abs(rr)), axis=0, keepdims=True)

    l1 = -0.25 * (1.0 - cg) * (1.0 - cg) * jnp.log(cg)
    l0 = -0.75 * cg * cg * jnp.log(1.0 - cg)
    cls_corr = jnp.sum(m_s * (l1 - l0))
    reg_corr = jnp.sum(m_s * cnt_s * (_smooth_l1(jnp.abs(1.0 - g))
                                      - _smooth_l1(jnp.abs(g))))

    acc_ref[0] = acc_ref[0] + (-0.75) * part + cls_corr
    acc_ref[1] = acc_ref[1] + reg_corr

    @pl.when(b == _B - 1)
    def _fin():
        sums_ref[0, 0] = acc_ref[0]
        sums_ref[0, 1] = acc_ref[1]


def _final_body(w_ref, scol_ref, sums_ref, out_cls_ref, out_reg_ref):
    reg_base = jnp.sum(w_ref[...] * scol_ref[...])
    out_cls_ref[0, 0] = sums_ref[0, 0] / float(_NV * _B)
    out_reg_ref[0, 0] = (reg_base + sums_ref[0, 1]) / float(_NV * _C * _B)


def _tc_call_kwargs():
    return dict(
        grid=(_B,),
        in_specs=[
            pl.BlockSpec((1, _C, _N), lambda b: (b, 0, 0)),
            pl.BlockSpec((1, _C, _N), lambda b: (b, 0, 0)),
            pl.BlockSpec((1, 3, _A), lambda b: (b, 0, 0)),
        ],
        out_specs=[
            pl.BlockSpec(memory_space=pltpu.SMEM),
            pl.BlockSpec((1, 1, _N), lambda b: (b, 0, 0)),
        ],
        out_shape=[
            jax.ShapeDtypeStruct((1, 2), jnp.float32),
            jax.ShapeDtypeStruct((_B, 1, _N), jnp.float32),
        ],
        scratch_shapes=[pltpu.SMEM((2,), jnp.float32)],
    )


def _final_call_kwargs():
    return dict(
        grid=(1,),
        in_specs=[
            pl.BlockSpec((_B, 1, _N), lambda i: (0, 0, 0)),
            pl.BlockSpec((_B, 1, _N), lambda i: (0, 0, 0)),
            pl.BlockSpec(memory_space=pltpu.SMEM),
        ],
        out_specs=[
            pl.BlockSpec(memory_space=pltpu.SMEM),
            pl.BlockSpec(memory_space=pltpu.SMEM),
        ],
        out_shape=[
            jax.ShapeDtypeStruct((1, 1), jnp.float32),
            jax.ShapeDtypeStruct((1, 1), jnp.float32),
        ],
    )


def kernel(classifications, regressions, annotations):
    ann_t = annotations.transpose(0, 2, 1)
    zeros_n = jnp.zeros((_N,), jnp.float32)
    w = _sc_scatter_counts(ann_t, zeros_n)

    ct = classifications.transpose(0, 2, 1)
    rt = regressions.transpose(0, 2, 1)
    sums, scol = pl.pallas_call(_tc_body, **_tc_call_kwargs())(ct, rt, ann_t)
    out_c, out_r = pl.pallas_call(_final_body, **_final_call_kwargs())(
        w, scol, sums)
    return out_c.reshape(1), out_r.reshape(1)
